# baseline (speedup 1.0000x reference)
import functools

import jax
import jax.numpy as jnp
from jax import lax
from jax.experimental import pallas as pl
from jax.experimental.pallas import tpu as pltpu
from jax.experimental.pallas import tpu_sc as plsc

HID = 128
NL = 3
NG = 256
N = 10000
E = 320000
NBT = 4
BN_EPS = 1e-5

NC = 2
NS = 16
NW = NC * NS
K = 128
NCH = 80
EPW = NCH * K
EPAD = NW * EPW
NBUF = 2
IB = 8
GARB = N + 48
NPAD = 10240
RPT = NPAD // NS

FP32 = jnp.float32
I32 = jnp.int32


def _gidx_body(src_ref, bt_ref, o_ref):
    o_ref[...] = src_ref[...] * NBT + jnp.clip(bt_ref[...], 0, NBT - 1)


def _compute_gidx(src2d, bt2d):
    return pl.pallas_call(
        _gidx_body,
        out_shape=jax.ShapeDtypeStruct(src2d.shape, I32),
    )(src2d, bt2d)


_EMB_SIZES = ((101, 64), (6, 16), (2, 8), (5, 8), (6, 8))


def _embed_body(x_ref, ea_ref, ed_ref, er_ref, ef_ref, eh_ref,
                pw_ref, pb_ref, wcat_ref, h_ref, ht_ref):
    xb = x_ref[...]
    bn = xb.shape[0]
    pieces = []
    for col, (rows, _), t_ref in zip(
            range(5), _EMB_SIZES, (ea_ref, ed_ref, er_ref, ef_ref, eh_ref)):
        idx = jnp.clip(xb[:, col:col + 1], 0, rows - 1)
        oh = (idx == lax.broadcasted_iota(I32, (bn, rows), 1)).astype(FP32)
        pieces.append(jnp.dot(oh, t_ref[...], preferred_element_type=FP32))
    hcat = jnp.concatenate(pieces, axis=1)
    h = jnp.dot(hcat, pw_ref[...], preferred_element_type=FP32) + pb_ref[...]
    h_ref[...] = h
    ht_ref[...] = jnp.dot(h, wcat_ref[...], preferred_element_type=FP32)


def _embed(x2d, embs, proj_w, proj_b2, wcat0):
    bn = 1000
    grid = N // bn
    in_specs = [pl.BlockSpec((bn, 5), lambda i: (i, 0))]
    in_specs += [pl.BlockSpec(t.shape, lambda i: (0, 0)) for t in embs]
    in_specs += [
        pl.BlockSpec(proj_w.shape, lambda i: (0, 0)),
        pl.BlockSpec(proj_b2.shape, lambda i: (0, 0)),
        pl.BlockSpec(wcat0.shape, lambda i: (0, 0)),
    ]
    return pl.pallas_call(
        _embed_body,
        grid=(grid,),
        in_specs=in_specs,
        out_specs=[
            pl.BlockSpec((bn, HID), lambda i: (i, 0)),
            pl.BlockSpec((bn, NBT * HID), lambda i: (i, 0)),
        ],
        out_shape=[
            jax.ShapeDtypeStruct((N, HID), FP32),
            jax.ShapeDtypeStruct((N, NBT * HID), FP32),
        ],
    )(x2d, *embs, proj_w, proj_b2, wcat0)


def _sc_body(ht_hbm, idx_hbm, zeros_hbm, out_hbm,
             idx_v, rows_v, m_sh, *sems):
    c = lax.axis_index("c")
    s = lax.axis_index("s")
    wid = c * NS + s
    gsem = sems[:NBUF]
    ssem = sems[NBUF:2 * NBUF]
    isem = sems[2 * NBUF:]

    pltpu.sync_copy(zeros_hbm, m_sh.at[pl.ds(s * RPT, RPT)])

    def idx_load(j, ib):
        pltpu.async_copy(idx_hbm.at[wid, pl.ds(2 * j, 2)],
                         idx_v.at[pl.ds(2 * ib, 2)], isem[ib])

    def idx_wait(j, ib):
        pltpu.make_async_copy(idx_hbm.at[wid, pl.ds(2 * j, 2)],
                              idx_v.at[pl.ds(2 * ib, 2)], isem[ib]).wait()

    def gather(ib, b):
        pltpu.async_copy(ht_hbm.at[idx_v.at[2 * ib]], rows_v.at[b], gsem[b])

    def gather_wait(ib, b):
        pltpu.make_async_copy(ht_hbm.at[idx_v.at[2 * ib]], rows_v.at[b],
                              gsem[b]).wait()

    def scatter(ib, b):
        pltpu.async_copy(rows_v.at[b], m_sh.at[idx_v.at[2 * ib + 1]],
                         ssem[b], add=True)

    def scatter_wait(ib, b):
        pltpu.make_async_copy(rows_v.at[b], m_sh.at[idx_v.at[2 * ib + 1]],
                              ssem[b]).wait()

    for j in range(6):
        idx_load(j, j)
    plsc.subcore_barrier()
    idx_wait(0, 0)
    gather(0, 0)

    def outer(j0, carry):
        for t in range(IB):
            j = j0 * IB + t
            b = t % NBUF
            ib = t
            gather_wait(ib, b)
            scatter(ib, b)
            b2 = (b + 1) % NBUF
            ib2 = (t + 1) % IB
            ibm1 = (t - 1) % IB
            @pl.when(j >= 1)
            def _():
                scatter_wait(ibm1, b2)
            @pl.when(j + 1 < NCH)
            def _():
                idx_wait(j + 1, ib2)
                gather(ib2, b2)
            @pl.when(j + 6 < NCH)
            def _():
                idx_load(j + 6, (t + 6) % IB)
        return carry

    lax.fori_loop(0, NCH // IB, outer, 0)
    scatter_wait((NCH - 1) % IB, (NCH - 1) % NBUF)
    plsc.subcore_barrier()

    pltpu.sync_copy(m_sh.at[pl.ds(s * RPT, RPT)],
                    out_hbm.at[c, pl.ds(s * RPT, RPT)])


def _sc_aggregate(ht4, idx_p, zeros):
    mesh = plsc.VectorSubcoreMesh(core_axis_name="c", subcore_axis_name="s")
    k = pl.kernel(
        _sc_body,
        out_type=jax.ShapeDtypeStruct((NC, NPAD, HID), FP32),
        mesh=mesh,
        scratch_types=[
            pltpu.VMEM((2 * IB, K), I32),
            pltpu.VMEM((NBUF, K, HID), FP32),
            pltpu.VMEM_SHARED((NPAD, HID), FP32),
        ] + [pltpu.SemaphoreType.DMA] * (2 * NBUF + IB),
    )
    return k(ht4, idx_p, zeros)


def _gru_body(mp_ref, h_ref, wih_ref, whh_ref, bih_ref, bhh_ref,
              wcat_ref, h_out, ht_out):
    m = mp_ref[0] + mp_ref[1]
    h = h_ref[...]
    gi = jnp.dot(m, wih_ref[...], preferred_element_type=FP32) + bih_ref[...]
    gh = jnp.dot(h, whh_ref[...], preferred_element_type=FP32) + bhh_ref[...]
    i_r, i_z, i_n = gi[:, :HID], gi[:, HID:2 * HID], gi[:, 2 * HID:]
    h_r, h_z, h_n = gh[:, :HID], gh[:, HID:2 * HID], gh[:, 2 * HID:]
    r = jax.nn.sigmoid(i_r + h_r)
    z = jax.nn.sigmoid(i_z + h_z)
    n = jnp.tanh(i_n + r * h_n)
    h_new = (1.0 - z) * n + z * h
    h_out[...] = h_new
    if ht_out is not None:
        ht_out[...] = jnp.dot(h_new, wcat_ref[...],
                              preferred_element_type=FP32)


def _gru(mp, h, wihT, whhT, bih2, bhh2, wcat_next):
    bn = 1000
    grid = N // bn
    last = wcat_next is None
    if last:
        wcat_next = jnp.zeros((HID, 8), FP32)

    body = functools.partial(_gru_body) if not last else (
        lambda mp_ref, h_ref, wih_ref, whh_ref, bih_ref, bhh_ref,
               wcat_ref, h_out:
        _gru_body(mp_ref, h_ref, wih_ref, whh_ref, bih_ref, bhh_ref,
                  wcat_ref, h_out, None))

    in_specs = [
        pl.BlockSpec((NC, bn, HID), lambda i: (0, i, 0)),
        pl.BlockSpec((bn, HID), lambda i: (i, 0)),
        pl.BlockSpec(wihT.shape, lambda i: (0, 0)),
        pl.BlockSpec(whhT.shape, lambda i: (0, 0)),
        pl.BlockSpec(bih2.shape, lambda i: (0, 0)),
        pl.BlockSpec(bhh2.shape, lambda i: (0, 0)),
        pl.BlockSpec(wcat_next.shape, lambda i: (0, 0)),
    ]
    if last:
        out_specs = pl.BlockSpec((bn, HID), lambda i: (i, 0))
        out_shape = jax.ShapeDtypeStruct((N, HID), FP32)
    else:
        out_specs = [
            pl.BlockSpec((bn, HID), lambda i: (i, 0)),
            pl.BlockSpec((bn, NBT * HID), lambda i: (i, 0)),
        ]
        out_shape = [
            jax.ShapeDtypeStruct((N, HID), FP32),
            jax.ShapeDtypeStruct((N, NBT * HID), FP32),
        ]
    return pl.pallas_call(
        body,
        grid=(grid,),
        in_specs=in_specs,
        out_specs=out_specs,
        out_shape=out_shape,
    )(mp, h, wihT, whhT, bih2, bhh2, wcat_next)


def _bn_eval(v, g, b):
    return g * (v / jnp.sqrt(1.0 + BN_EPS)) + b


def _readout_body(h_ref, batch_ref, fpw_ref, fpb_ref,
                  fc1w_ref, fc1b_ref, bn1g_ref, bn1b_ref,
                  fc2w_ref, fc2b_ref, bn2g_ref, bn2b_ref,
                  ow_ref, ob_ref, o_ref):
    chunks = batch_ref.shape[0]
    bn = batch_ref.shape[1]
    gids = lax.broadcasted_iota(I32, (NG, 1), 0)
    hs = jnp.zeros((NG, HID), FP32)
    cnt = jnp.zeros((NG, 1), FP32)
    for j in range(chunks):
        bj = batch_ref[j:j + 1, :]
        oh = (bj == gids).astype(FP32)
        hj = h_ref[pl.ds(j * bn, bn), :]
        hs = hs + jnp.dot(oh, hj, preferred_element_type=FP32)
        cnt = cnt + jnp.sum(oh, axis=1, keepdims=True)
    g = jnp.dot(hs, fpw_ref[...], preferred_element_type=FP32) \
        + cnt * fpb_ref[...]
    z1 = jax.nn.relu(_bn_eval(
        jnp.dot(g, fc1w_ref[...], preferred_element_type=FP32)
        + fc1b_ref[...], bn1g_ref[...], bn1b_ref[...]))
    z2 = jax.nn.relu(_bn_eval(
        jnp.dot(z1, fc2w_ref[...], preferred_element_type=FP32)
        + fc2b_ref[...], bn2g_ref[...], bn2b_ref[...]))
    o_ref[...] = jnp.dot(z2, ow_ref[...], preferred_element_type=FP32) \
        + ob_ref[...]


def _readout(h, batch2d, p):
    args = (
        h, batch2d,
        p['fp_w'], p['fp_b'].reshape(1, -1),
        p['fc1_w'], p['fc1_b'].reshape(1, -1),
        p['bn1_g'].reshape(1, -1), p['bn1_b'].reshape(1, -1),
        p['fc2_w'], p['fc2_b'].reshape(1, -1),
        p['bn2_g'].reshape(1, -1), p['bn2_b'].reshape(1, -1),
        p['out_w'], p['out_b'].reshape(1, -1),
    )
    return pl.pallas_call(
        _readout_body,
        out_shape=jax.ShapeDtypeStruct((NG, p['out_w'].shape[1]), FP32),
    )(*args)


def _wcat(W):
    return jnp.transpose(W, (2, 0, 1)).reshape(HID, NBT * HID)


def kernel(x, edge_index, edge_attr, batch, params):
    x = x.astype(I32)
    src = edge_index[0].astype(I32)
    dst = edge_index[1].astype(I32)
    bt = edge_attr[:, 0].astype(I32)
    batch2d = batch.astype(I32).reshape(10, 1000)

    p = params
    embs = (p['emb_atomic'], p['emb_degree'], p['emb_aroma'],
            p['emb_fc'], p['emb_hyb'])
    wcats = [_wcat(lp['W']) for lp in p['layers']]

    gidx_f = _compute_gidx(src.reshape(2500, 128),
                           bt.reshape(2500, 128)).reshape(E)
    gidx_p = jnp.concatenate([gidx_f, jnp.zeros((EPAD - E,), I32)])
    dst_p = jnp.concatenate([dst, jnp.full((EPAD - E,), GARB, I32)])
    idx_p = jnp.stack([gidx_p.reshape(NW, NCH, K),
                       dst_p.reshape(NW, NCH, K)],
                      axis=2).reshape(NW, 2 * NCH, K)

    h, ht = _embed(x, embs, p['node_proj_w'],
                   p['node_proj_b'].reshape(1, HID), wcats[0])

    zeros = jnp.zeros((RPT, HID), FP32)
    for l, lp in enumerate(p['layers']):
        ht4 = ht.reshape(NBT * N, HID)
        mp = _sc_aggregate(ht4, idx_p, zeros)
        wcat_next = wcats[l + 1] if l + 1 < NL else None
        res = _gru(mp, h,
                   lp['W_ih'].T, lp['W_hh'].T,
                   lp['b_ih'].reshape(1, -1), lp['b_hh'].reshape(1, -1),
                   wcat_next)
        if wcat_next is None:
            h = res
            ht = None
        else:
            h, ht = res

    return _readout(h, batch2d, p)


# speedup vs baseline: 2.9526x; 2.9526x over previous
import functools

import jax
import jax.numpy as jnp
from jax import lax
from jax.experimental import pallas as pl
from jax.experimental.pallas import tpu as pltpu
from jax.experimental.pallas import tpu_sc as plsc

HID = 128
NL = 3
NG = 256
N = 10000
E = 320000
NBT = 4
BN_EPS = 1e-5

NC = 2
NS = 16
NW = NC * NS
K = 50
NCH = 200
EPW = NCH * K
EPAD = NW * EPW
NBUF = 5
GLEAD = 3
IB = 10
GARB = N + 48
NPAD = 10240
RPT = NPAD // NS

FP32 = jnp.float32
I32 = jnp.int32


def _gidx_body(src_ref, bt_ref, o_ref):
    o_ref[...] = src_ref[...] * NBT + jnp.clip(bt_ref[...], 0, NBT - 1)


def _compute_gidx(src2d, bt2d):
    return pl.pallas_call(
        _gidx_body,
        out_shape=jax.ShapeDtypeStruct(src2d.shape, I32),
    )(src2d, bt2d)


_EMB_SIZES = ((101, 64), (6, 16), (2, 8), (5, 8), (6, 8))


def _embed_body(x_ref, ea_ref, ed_ref, er_ref, ef_ref, eh_ref,
                pw_ref, pb_ref, wcat_ref, h_ref, ht_ref):
    xb = x_ref[...]
    bn = xb.shape[0]
    pieces = []
    for col, (rows, _), t_ref in zip(
            range(5), _EMB_SIZES, (ea_ref, ed_ref, er_ref, ef_ref, eh_ref)):
        idx = jnp.clip(xb[:, col:col + 1], 0, rows - 1)
        oh = (idx == lax.broadcasted_iota(I32, (bn, rows), 1)).astype(FP32)
        pieces.append(jnp.dot(oh, t_ref[...], preferred_element_type=FP32))
    hcat = jnp.concatenate(pieces, axis=1)
    h = jnp.dot(hcat, pw_ref[...], preferred_element_type=FP32) + pb_ref[...]
    h_ref[...] = h
    ht_ref[...] = jnp.dot(h, wcat_ref[...], preferred_element_type=FP32)


def _embed(x2d, embs, proj_w, proj_b2, wcat0):
    bn = 1000
    grid = N // bn
    in_specs = [pl.BlockSpec((bn, 5), lambda i: (i, 0))]
    in_specs += [pl.BlockSpec(t.shape, lambda i: (0, 0)) for t in embs]
    in_specs += [
        pl.BlockSpec(proj_w.shape, lambda i: (0, 0)),
        pl.BlockSpec(proj_b2.shape, lambda i: (0, 0)),
        pl.BlockSpec(wcat0.shape, lambda i: (0, 0)),
    ]
    return pl.pallas_call(
        _embed_body,
        grid=(grid,),
        in_specs=in_specs,
        out_specs=[
            pl.BlockSpec((bn, HID), lambda i: (i, 0)),
            pl.BlockSpec((bn, NBT * HID), lambda i: (i, 0)),
        ],
        out_shape=[
            jax.ShapeDtypeStruct((N, HID), FP32),
            jax.ShapeDtypeStruct((N, NBT * HID), FP32),
        ],
    )(x2d, *embs, proj_w, proj_b2, wcat0)


def _sc_body(ht_hbm, idx_hbm, zeros_hbm, out_hbm,
             idx_v, rows_v, m_sh, *sems):
    c = lax.axis_index("c")
    s = lax.axis_index("s")
    wid = c * NS + s
    gsem = sems[:NBUF]
    ssem = sems[NBUF:2 * NBUF]
    isem = sems[2 * NBUF:]

    pltpu.sync_copy(zeros_hbm, m_sh.at[pl.ds(s * RPT, RPT)])

    def idx_load(j, ib):
        pltpu.async_copy(idx_hbm.at[wid, pl.ds(2 * j, 2)],
                         idx_v.at[pl.ds(2 * ib, 2)], isem[ib])

    def idx_wait(j, ib):
        pltpu.make_async_copy(idx_hbm.at[wid, pl.ds(2 * j, 2)],
                              idx_v.at[pl.ds(2 * ib, 2)], isem[ib]).wait()

    def gather(ib, b):
        pltpu.async_copy(ht_hbm.at[idx_v.at[2 * ib]], rows_v.at[b], gsem[b])

    def gather_wait(ib, b):
        pltpu.make_async_copy(ht_hbm.at[idx_v.at[2 * ib]], rows_v.at[b],
                              gsem[b]).wait()

    def scatter(ib, b):
        pltpu.async_copy(rows_v.at[b], m_sh.at[idx_v.at[2 * ib + 1]],
                         ssem[b], add=True)

    def scatter_wait(ib, b):
        pltpu.make_async_copy(rows_v.at[b], m_sh.at[idx_v.at[2 * ib + 1]],
                              ssem[b]).wait()

    for j in range(IB - 2):
        idx_load(j, j)
    plsc.subcore_barrier()
    for j in range(GLEAD):
        idx_wait(j, j)
        gather(j, j)

    def outer(j0, carry):
        for t in range(IB):
            j = j0 * IB + t
            b = t % NBUF
            ib = t
            gather_wait(ib, b)
            scatter(ib, b)
            b2 = (b + GLEAD) % NBUF
            ib2 = (t + GLEAD) % IB
            ibm2 = (t - 2) % IB
            @pl.when(j >= 2)
            def _():
                scatter_wait(ibm2, b2)
            @pl.when(j + GLEAD < NCH)
            def _():
                idx_wait(j + GLEAD, ib2)
                gather(ib2, b2)
            @pl.when(j + IB - 2 < NCH)
            def _():
                idx_load(j + IB - 2, (t + IB - 2) % IB)
        return carry

    lax.fori_loop(0, NCH // IB, outer, 0)
    scatter_wait((NCH - 2) % IB, (NCH - 2) % NBUF)
    scatter_wait((NCH - 1) % IB, (NCH - 1) % NBUF)
    plsc.subcore_barrier()

    pltpu.sync_copy(m_sh.at[pl.ds(s * RPT, RPT)],
                    out_hbm.at[c, pl.ds(s * RPT, RPT)])


def _sc_aggregate(ht4, idx_p, zeros):
    mesh = plsc.VectorSubcoreMesh(core_axis_name="c", subcore_axis_name="s")
    k = pl.kernel(
        _sc_body,
        out_type=jax.ShapeDtypeStruct((NC, NPAD, HID), FP32),
        mesh=mesh,
        scratch_types=[
            pltpu.VMEM((2 * IB, K), I32),
            pltpu.VMEM((NBUF, K, HID), FP32),
            pltpu.VMEM_SHARED((NPAD, HID), FP32),
        ] + [pltpu.SemaphoreType.DMA] * (2 * NBUF + IB),
        name="sc_edge_aggregate",
    )
    return k(ht4, idx_p, zeros)


def _gru_body(mp_ref, h_ref, wih_ref, whh_ref, bih_ref, bhh_ref,
              wcat_ref, h_out, ht_out):
    m = mp_ref[0] + mp_ref[1]
    h = h_ref[...]
    gi = jnp.dot(m, wih_ref[...], preferred_element_type=FP32) + bih_ref[...]
    gh = jnp.dot(h, whh_ref[...], preferred_element_type=FP32) + bhh_ref[...]
    i_r, i_z, i_n = gi[:, :HID], gi[:, HID:2 * HID], gi[:, 2 * HID:]
    h_r, h_z, h_n = gh[:, :HID], gh[:, HID:2 * HID], gh[:, 2 * HID:]
    r = jax.nn.sigmoid(i_r + h_r)
    z = jax.nn.sigmoid(i_z + h_z)
    n = jnp.tanh(i_n + r * h_n)
    h_new = (1.0 - z) * n + z * h
    h_out[...] = h_new
    if ht_out is not None:
        ht_out[...] = jnp.dot(h_new, wcat_ref[...],
                              preferred_element_type=FP32)


def _gru(mp, h, wihT, whhT, bih2, bhh2, wcat_next):
    bn = 1000
    grid = N // bn
    last = wcat_next is None
    if last:
        wcat_next = jnp.zeros((HID, 8), FP32)

    body = functools.partial(_gru_body) if not last else (
        lambda mp_ref, h_ref, wih_ref, whh_ref, bih_ref, bhh_ref,
               wcat_ref, h_out:
        _gru_body(mp_ref, h_ref, wih_ref, whh_ref, bih_ref, bhh_ref,
                  wcat_ref, h_out, None))

    in_specs = [
        pl.BlockSpec((NC, bn, HID), lambda i: (0, i, 0)),
        pl.BlockSpec((bn, HID), lambda i: (i, 0)),
        pl.BlockSpec(wihT.shape, lambda i: (0, 0)),
        pl.BlockSpec(whhT.shape, lambda i: (0, 0)),
        pl.BlockSpec(bih2.shape, lambda i: (0, 0)),
        pl.BlockSpec(bhh2.shape, lambda i: (0, 0)),
        pl.BlockSpec(wcat_next.shape, lambda i: (0, 0)),
    ]
    if last:
        out_specs = pl.BlockSpec((bn, HID), lambda i: (i, 0))
        out_shape = jax.ShapeDtypeStruct((N, HID), FP32)
    else:
        out_specs = [
            pl.BlockSpec((bn, HID), lambda i: (i, 0)),
            pl.BlockSpec((bn, NBT * HID), lambda i: (i, 0)),
        ]
        out_shape = [
            jax.ShapeDtypeStruct((N, HID), FP32),
            jax.ShapeDtypeStruct((N, NBT * HID), FP32),
        ]
    return pl.pallas_call(
        body,
        grid=(grid,),
        in_specs=in_specs,
        out_specs=out_specs,
        out_shape=out_shape,
    )(mp, h, wihT, whhT, bih2, bhh2, wcat_next)


def _bn_eval(v, g, b):
    return g * (v / jnp.sqrt(1.0 + BN_EPS)) + b


def _readout_body(h_ref, batch_ref, fpw_ref, fpb_ref,
                  fc1w_ref, fc1b_ref, bn1g_ref, bn1b_ref,
                  fc2w_ref, fc2b_ref, bn2g_ref, bn2b_ref,
                  ow_ref, ob_ref, o_ref):
    chunks = batch_ref.shape[0]
    bn = batch_ref.shape[1]
    gids = lax.broadcasted_iota(I32, (NG, 1), 0)
    hs = jnp.zeros((NG, HID), FP32)
    cnt = jnp.zeros((NG, 1), FP32)
    for j in range(chunks):
        bj = batch_ref[j:j + 1, :]
        oh = (bj == gids).astype(FP32)
        hj = h_ref[pl.ds(j * bn, bn), :]
        hs = hs + jnp.dot(oh, hj, preferred_element_type=FP32)
        cnt = cnt + jnp.sum(oh, axis=1, keepdims=True)
    g = jnp.dot(hs, fpw_ref[...], preferred_element_type=FP32) \
        + cnt * fpb_ref[...]
    z1 = jax.nn.relu(_bn_eval(
        jnp.dot(g, fc1w_ref[...], preferred_element_type=FP32)
        + fc1b_ref[...], bn1g_ref[...], bn1b_ref[...]))
    z2 = jax.nn.relu(_bn_eval(
        jnp.dot(z1, fc2w_ref[...], preferred_element_type=FP32)
        + fc2b_ref[...], bn2g_ref[...], bn2b_ref[...]))
    o_ref[...] = jnp.dot(z2, ow_ref[...], preferred_element_type=FP32) \
        + ob_ref[...]


def _readout(h, batch2d, p):
    args = (
        h, batch2d,
        p['fp_w'], p['fp_b'].reshape(1, -1),
        p['fc1_w'], p['fc1_b'].reshape(1, -1),
        p['bn1_g'].reshape(1, -1), p['bn1_b'].reshape(1, -1),
        p['fc2_w'], p['fc2_b'].reshape(1, -1),
        p['bn2_g'].reshape(1, -1), p['bn2_b'].reshape(1, -1),
        p['out_w'], p['out_b'].reshape(1, -1),
    )
    return pl.pallas_call(
        _readout_body,
        out_shape=jax.ShapeDtypeStruct((NG, p['out_w'].shape[1]), FP32),
    )(*args)


def _wcat(W):
    return jnp.transpose(W, (2, 0, 1)).reshape(HID, NBT * HID)


def kernel(x, edge_index, edge_attr, batch, params):
    x = x.astype(I32)
    src = edge_index[0].astype(I32)
    dst = edge_index[1].astype(I32)
    bt = edge_attr[:, 0].astype(I32)
    batch2d = batch.astype(I32).reshape(10, 1000)

    p = params
    embs = (p['emb_atomic'], p['emb_degree'], p['emb_aroma'],
            p['emb_fc'], p['emb_hyb'])
    wcats = [_wcat(lp['W']) for lp in p['layers']]

    gidx_f = _compute_gidx(src.reshape(2500, 128),
                           bt.reshape(2500, 128)).reshape(E)
    gidx_p = jnp.concatenate([gidx_f, jnp.zeros((EPAD - E,), I32)])
    dst_p = jnp.concatenate([dst, jnp.full((EPAD - E,), GARB, I32)])
    idx_p = jnp.stack([gidx_p.reshape(NW, NCH, K),
                       dst_p.reshape(NW, NCH, K)],
                      axis=2).reshape(NW, 2 * NCH, K)

    h, ht = _embed(x, embs, p['node_proj_w'],
                   p['node_proj_b'].reshape(1, HID), wcats[0])

    zeros = jnp.zeros((RPT, HID), FP32)
    for l, lp in enumerate(p['layers']):
        ht4 = ht.reshape(NBT * N, HID)
        mp = _sc_aggregate(ht4, idx_p, zeros)
        wcat_next = wcats[l + 1] if l + 1 < NL else None
        res = _gru(mp, h,
                   lp['W_ih'].T, lp['W_hh'].T,
                   lp['b_ih'].reshape(1, -1), lp['b_hh'].reshape(1, -1),
                   wcat_next)
        if wcat_next is None:
            h = res
            ht = None
        else:
            h, ht = res

    return _readout(h, batch2d, p)


# speedup vs baseline: 3.0626x; 1.0373x over previous
import functools

import jax
import jax.numpy as jnp
from jax import lax
from jax.experimental import pallas as pl
from jax.experimental.pallas import tpu as pltpu
from jax.experimental.pallas import tpu_sc as plsc

HID = 128
NL = 3
NG = 256
N = 10000
E = 320000
NBT = 4
BN_EPS = 1e-5

NC = 2
NS = 16
NW = NC * NS
K = 50
NCH = 200
EPW = NCH * K
EPAD = NW * EPW
NBUF = 5
GLEAD = 4
SWAIT = NBUF - GLEAD
IB = 10
GARB = N + 48
NPAD = 10240
RPT = NPAD // NS

FP32 = jnp.float32
I32 = jnp.int32


def _gidx_body(src_ref, bt_ref, o_ref):
    o_ref[...] = src_ref[...] * NBT + jnp.clip(bt_ref[...], 0, NBT - 1)


def _compute_gidx(src2d, bt2d):
    return pl.pallas_call(
        _gidx_body,
        out_shape=jax.ShapeDtypeStruct(src2d.shape, I32),
    )(src2d, bt2d)


_EMB_SIZES = ((101, 64), (6, 16), (2, 8), (5, 8), (6, 8))


def _embed_body(x_ref, ea_ref, ed_ref, er_ref, ef_ref, eh_ref,
                pw_ref, pb_ref, wcat_ref, h_ref, ht_ref):
    xb = x_ref[...]
    bn = xb.shape[0]
    pieces = []
    for col, (rows, _), t_ref in zip(
            range(5), _EMB_SIZES, (ea_ref, ed_ref, er_ref, ef_ref, eh_ref)):
        idx = jnp.clip(xb[:, col:col + 1], 0, rows - 1)
        oh = (idx == lax.broadcasted_iota(I32, (bn, rows), 1)).astype(FP32)
        pieces.append(jnp.dot(oh, t_ref[...], preferred_element_type=FP32))
    hcat = jnp.concatenate(pieces, axis=1)
    h = jnp.dot(hcat, pw_ref[...], preferred_element_type=FP32) + pb_ref[...]
    h_ref[...] = h
    ht_ref[...] = jnp.dot(h, wcat_ref[...], preferred_element_type=FP32)


def _embed(x2d, embs, proj_w, proj_b2, wcat0):
    bn = 1000
    grid = N // bn
    in_specs = [pl.BlockSpec((bn, 5), lambda i: (i, 0))]
    in_specs += [pl.BlockSpec(t.shape, lambda i: (0, 0)) for t in embs]
    in_specs += [
        pl.BlockSpec(proj_w.shape, lambda i: (0, 0)),
        pl.BlockSpec(proj_b2.shape, lambda i: (0, 0)),
        pl.BlockSpec(wcat0.shape, lambda i: (0, 0)),
    ]
    return pl.pallas_call(
        _embed_body,
        grid=(grid,),
        in_specs=in_specs,
        out_specs=[
            pl.BlockSpec((bn, HID), lambda i: (i, 0)),
            pl.BlockSpec((bn, NBT * HID), lambda i: (i, 0)),
        ],
        out_shape=[
            jax.ShapeDtypeStruct((N, HID), FP32),
            jax.ShapeDtypeStruct((N, NBT * HID), FP32),
        ],
    )(x2d, *embs, proj_w, proj_b2, wcat0)


def _sc_body(ht_hbm, idx_hbm, zeros_hbm, out_hbm,
             idx_v, rows_v, m_sh, *sems):
    c = lax.axis_index("c")
    s = lax.axis_index("s")
    wid = c * NS + s
    gsem = sems[:NBUF]
    ssem = sems[NBUF:2 * NBUF]
    isem = sems[2 * NBUF:]

    pltpu.sync_copy(zeros_hbm, m_sh.at[pl.ds(s * RPT, RPT)])

    def idx_load(j, ib):
        pltpu.async_copy(idx_hbm.at[wid, pl.ds(2 * j, 2)],
                         idx_v.at[pl.ds(2 * ib, 2)], isem[ib])

    def idx_wait(j, ib):
        pltpu.make_async_copy(idx_hbm.at[wid, pl.ds(2 * j, 2)],
                              idx_v.at[pl.ds(2 * ib, 2)], isem[ib]).wait()

    def gather(ib, b):
        pltpu.async_copy(ht_hbm.at[idx_v.at[2 * ib]], rows_v.at[b], gsem[b])

    def gather_wait(ib, b):
        pltpu.make_async_copy(ht_hbm.at[idx_v.at[2 * ib]], rows_v.at[b],
                              gsem[b]).wait()

    def scatter(ib, b):
        pltpu.async_copy(rows_v.at[b], m_sh.at[idx_v.at[2 * ib + 1]],
                         ssem[b], add=True)

    def scatter_wait(ib, b):
        pltpu.make_async_copy(rows_v.at[b], m_sh.at[idx_v.at[2 * ib + 1]],
                              ssem[b]).wait()

    for j in range(IB - 2):
        idx_load(j, j)
    plsc.subcore_barrier()
    for j in range(GLEAD):
        idx_wait(j, j)
        gather(j, j)

    def outer(j0, carry):
        for t in range(IB):
            j = j0 * IB + t
            b = t % NBUF
            ib = t
            gather_wait(ib, b)
            scatter(ib, b)
            b2 = (b + GLEAD) % NBUF
            ib2 = (t + GLEAD) % IB
            ibm2 = (t - SWAIT) % IB
            @pl.when(j >= SWAIT)
            def _():
                scatter_wait(ibm2, b2)
            @pl.when(j + GLEAD < NCH)
            def _():
                idx_wait(j + GLEAD, ib2)
                gather(ib2, b2)
            @pl.when(j + IB - 2 < NCH)
            def _():
                idx_load(j + IB - 2, (t + IB - 2) % IB)
        return carry

    lax.fori_loop(0, NCH // IB, outer, 0)
    for d in range(SWAIT):
        jd = NCH - SWAIT + d
        scatter_wait(jd % IB, jd % NBUF)
    plsc.subcore_barrier()

    pltpu.sync_copy(m_sh.at[pl.ds(s * RPT, RPT)],
                    out_hbm.at[c, pl.ds(s * RPT, RPT)])


def _sc_aggregate(ht4, idx_p, zeros):
    mesh = plsc.VectorSubcoreMesh(core_axis_name="c", subcore_axis_name="s")
    k = pl.kernel(
        _sc_body,
        out_type=jax.ShapeDtypeStruct((NC, NPAD, HID), FP32),
        mesh=mesh,
        scratch_types=[
            pltpu.VMEM((2 * IB, K), I32),
            pltpu.VMEM((NBUF, K, HID), FP32),
            pltpu.VMEM_SHARED((NPAD, HID), FP32),
        ] + [pltpu.SemaphoreType.DMA] * (2 * NBUF + IB),
        name="sc_edge_aggregate",
    )
    return k(ht4, idx_p, zeros)


def _gru_body(mp_ref, h_ref, wih_ref, whh_ref, bih_ref, bhh_ref,
              wcat_ref, h_out, ht_out):
    m = mp_ref[0] + mp_ref[1]
    h = h_ref[...]
    gi = jnp.dot(m, wih_ref[...], preferred_element_type=FP32) + bih_ref[...]
    gh = jnp.dot(h, whh_ref[...], preferred_element_type=FP32) + bhh_ref[...]
    i_r, i_z, i_n = gi[:, :HID], gi[:, HID:2 * HID], gi[:, 2 * HID:]
    h_r, h_z, h_n = gh[:, :HID], gh[:, HID:2 * HID], gh[:, 2 * HID:]
    r = jax.nn.sigmoid(i_r + h_r)
    z = jax.nn.sigmoid(i_z + h_z)
    n = jnp.tanh(i_n + r * h_n)
    h_new = (1.0 - z) * n + z * h
    h_out[...] = h_new
    if ht_out is not None:
        ht_out[...] = jnp.dot(h_new, wcat_ref[...],
                              preferred_element_type=FP32)


def _gru(mp, h, wihT, whhT, bih2, bhh2, wcat_next):
    bn = 1000
    grid = N // bn
    last = wcat_next is None
    if last:
        wcat_next = jnp.zeros((HID, 8), FP32)

    body = functools.partial(_gru_body) if not last else (
        lambda mp_ref, h_ref, wih_ref, whh_ref, bih_ref, bhh_ref,
               wcat_ref, h_out:
        _gru_body(mp_ref, h_ref, wih_ref, whh_ref, bih_ref, bhh_ref,
                  wcat_ref, h_out, None))

    in_specs = [
        pl.BlockSpec((NC, bn, HID), lambda i: (0, i, 0)),
        pl.BlockSpec((bn, HID), lambda i: (i, 0)),
        pl.BlockSpec(wihT.shape, lambda i: (0, 0)),
        pl.BlockSpec(whhT.shape, lambda i: (0, 0)),
        pl.BlockSpec(bih2.shape, lambda i: (0, 0)),
        pl.BlockSpec(bhh2.shape, lambda i: (0, 0)),
        pl.BlockSpec(wcat_next.shape, lambda i: (0, 0)),
    ]
    if last:
        out_specs = pl.BlockSpec((bn, HID), lambda i: (i, 0))
        out_shape = jax.ShapeDtypeStruct((N, HID), FP32)
    else:
        out_specs = [
            pl.BlockSpec((bn, HID), lambda i: (i, 0)),
            pl.BlockSpec((bn, NBT * HID), lambda i: (i, 0)),
        ]
        out_shape = [
            jax.ShapeDtypeStruct((N, HID), FP32),
            jax.ShapeDtypeStruct((N, NBT * HID), FP32),
        ]
    return pl.pallas_call(
        body,
        grid=(grid,),
        in_specs=in_specs,
        out_specs=out_specs,
        out_shape=out_shape,
    )(mp, h, wihT, whhT, bih2, bhh2, wcat_next)


def _bn_eval(v, g, b):
    return g * (v / jnp.sqrt(1.0 + BN_EPS)) + b


def _readout_body(h_ref, batch_ref, fpw_ref, fpb_ref,
                  fc1w_ref, fc1b_ref, bn1g_ref, bn1b_ref,
                  fc2w_ref, fc2b_ref, bn2g_ref, bn2b_ref,
                  ow_ref, ob_ref, o_ref):
    chunks = batch_ref.shape[0]
    bn = batch_ref.shape[1]
    gids = lax.broadcasted_iota(I32, (NG, 1), 0)
    hs = jnp.zeros((NG, HID), FP32)
    cnt = jnp.zeros((NG, 1), FP32)
    for j in range(chunks):
        bj = batch_ref[j:j + 1, :]
        oh = (bj == gids).astype(FP32)
        hj = h_ref[pl.ds(j * bn, bn), :]
        hs = hs + jnp.dot(oh, hj, preferred_element_type=FP32)
        cnt = cnt + jnp.sum(oh, axis=1, keepdims=True)
    g = jnp.dot(hs, fpw_ref[...], preferred_element_type=FP32) \
        + cnt * fpb_ref[...]
    z1 = jax.nn.relu(_bn_eval(
        jnp.dot(g, fc1w_ref[...], preferred_element_type=FP32)
        + fc1b_ref[...], bn1g_ref[...], bn1b_ref[...]))
    z2 = jax.nn.relu(_bn_eval(
        jnp.dot(z1, fc2w_ref[...], preferred_element_type=FP32)
        + fc2b_ref[...], bn2g_ref[...], bn2b_ref[...]))
    o_ref[...] = jnp.dot(z2, ow_ref[...], preferred_element_type=FP32) \
        + ob_ref[...]


def _readout(h, batch2d, p):
    args = (
        h, batch2d,
        p['fp_w'], p['fp_b'].reshape(1, -1),
        p['fc1_w'], p['fc1_b'].reshape(1, -1),
        p['bn1_g'].reshape(1, -1), p['bn1_b'].reshape(1, -1),
        p['fc2_w'], p['fc2_b'].reshape(1, -1),
        p['bn2_g'].reshape(1, -1), p['bn2_b'].reshape(1, -1),
        p['out_w'], p['out_b'].reshape(1, -1),
    )
    return pl.pallas_call(
        _readout_body,
        out_shape=jax.ShapeDtypeStruct((NG, p['out_w'].shape[1]), FP32),
    )(*args)


def _wcat(W):
    return jnp.transpose(W, (2, 0, 1)).reshape(HID, NBT * HID)


def kernel(x, edge_index, edge_attr, batch, params):
    x = x.astype(I32)
    src = edge_index[0].astype(I32)
    dst = edge_index[1].astype(I32)
    bt = edge_attr[:, 0].astype(I32)
    batch2d = batch.astype(I32).reshape(10, 1000)

    p = params
    embs = (p['emb_atomic'], p['emb_degree'], p['emb_aroma'],
            p['emb_fc'], p['emb_hyb'])
    wcats = [_wcat(lp['W']) for lp in p['layers']]

    gidx_f = _compute_gidx(src.reshape(2500, 128),
                           bt.reshape(2500, 128)).reshape(E)
    gidx_p = jnp.concatenate([gidx_f, jnp.zeros((EPAD - E,), I32)])
    dst_p = jnp.concatenate([dst, jnp.full((EPAD - E,), GARB, I32)])
    idx_p = jnp.stack([gidx_p.reshape(NW, NCH, K),
                       dst_p.reshape(NW, NCH, K)],
                      axis=2).reshape(NW, 2 * NCH, K)

    h, ht = _embed(x, embs, p['node_proj_w'],
                   p['node_proj_b'].reshape(1, HID), wcats[0])

    zeros = jnp.zeros((RPT, HID), FP32)
    for l, lp in enumerate(p['layers']):
        ht4 = ht.reshape(NBT * N, HID)
        mp = _sc_aggregate(ht4, idx_p, zeros)
        wcat_next = wcats[l + 1] if l + 1 < NL else None
        res = _gru(mp, h,
                   lp['W_ih'].T, lp['W_hh'].T,
                   lp['b_ih'].reshape(1, -1), lp['b_hh'].reshape(1, -1),
                   wcat_next)
        if wcat_next is None:
            h = res
            ht = None
        else:
            h, ht = res

    return _readout(h, batch2d, p)


# speedup vs baseline: 3.1158x; 1.0173x over previous
import functools

import jax
import jax.numpy as jnp
from jax import lax
from jax.experimental import pallas as pl
from jax.experimental.pallas import tpu as pltpu
from jax.experimental.pallas import tpu_sc as plsc

HID = 128
NL = 3
NG = 256
N = 10000
E = 320000
NBT = 4
BN_EPS = 1e-5

NC = 2
NS = 16
NW = NC * NS
K = 50
NCH = 200
EPW = NCH * K
EPAD = NW * EPW
NBUF = 5
GLEAD = 4
SWAIT = NBUF - GLEAD
IB = 10
GARB = N + 48
NPAD = 10240
RPT = NPAD // NS

FP32 = jnp.float32
I32 = jnp.int32


_EMB_SIZES = ((101, 64), (6, 16), (2, 8), (5, 8), (6, 8))


def _embed_body(x_ref, src_ref, bt_ref, ea_ref, ed_ref, er_ref, ef_ref,
                eh_ref, pw_ref, pb_ref, wcat_ref, h_ref, ht_ref, gi_ref):
    xb = x_ref[...]
    bn = xb.shape[0]
    pieces = []
    for col, (rows, _), t_ref in zip(
            range(5), _EMB_SIZES, (ea_ref, ed_ref, er_ref, ef_ref, eh_ref)):
        idx = jnp.clip(xb[:, col:col + 1], 0, rows - 1)
        oh = (idx == lax.broadcasted_iota(I32, (bn, rows), 1)).astype(FP32)
        pieces.append(jnp.dot(oh, t_ref[...], preferred_element_type=FP32))
    hcat = jnp.concatenate(pieces, axis=1)
    h = jnp.dot(hcat, pw_ref[...], preferred_element_type=FP32) + pb_ref[...]
    h_ref[...] = h
    ht_ref[...] = jnp.dot(h, wcat_ref[...], preferred_element_type=FP32)
    gi_ref[...] = src_ref[...] * NBT + jnp.clip(bt_ref[...], 0, NBT - 1)


def _embed(x2d, src2d, bt2d, embs, proj_w, proj_b2, wcat0):
    bn = 1000
    grid = N // bn
    eb = src2d.shape[0] // grid
    in_specs = [
        pl.BlockSpec((bn, 5), lambda i: (i, 0)),
        pl.BlockSpec((eb, 160), lambda i: (i, 0)),
        pl.BlockSpec((eb, 160), lambda i: (i, 0)),
    ]
    in_specs += [pl.BlockSpec(t.shape, lambda i: (0, 0)) for t in embs]
    in_specs += [
        pl.BlockSpec(proj_w.shape, lambda i: (0, 0)),
        pl.BlockSpec(proj_b2.shape, lambda i: (0, 0)),
        pl.BlockSpec(wcat0.shape, lambda i: (0, 0)),
    ]
    return pl.pallas_call(
        _embed_body,
        grid=(grid,),
        in_specs=in_specs,
        out_specs=[
            pl.BlockSpec((bn, HID), lambda i: (i, 0)),
            pl.BlockSpec((bn, NBT * HID), lambda i: (i, 0)),
            pl.BlockSpec((eb, 160), lambda i: (i, 0)),
        ],
        out_shape=[
            jax.ShapeDtypeStruct((N, HID), FP32),
            jax.ShapeDtypeStruct((N, NBT * HID), FP32),
            jax.ShapeDtypeStruct(src2d.shape, I32),
        ],
    )(x2d, src2d, bt2d, *embs, proj_w, proj_b2, wcat0)


def _sc_body(ht_hbm, idx_hbm, zeros_hbm, out_hbm,
             idx_v, rows_v, m_sh, *sems):
    c = lax.axis_index("c")
    s = lax.axis_index("s")
    wid = c * NS + s
    gsem = sems[:NBUF]
    ssem = sems[NBUF:2 * NBUF]
    isem = sems[2 * NBUF:]

    pltpu.sync_copy(zeros_hbm, m_sh.at[pl.ds(s * RPT, RPT)])

    def idx_load(j, ib):
        pltpu.async_copy(idx_hbm.at[wid, pl.ds(2 * j, 2)],
                         idx_v.at[pl.ds(2 * ib, 2)], isem[ib])

    def idx_wait(j, ib):
        pltpu.make_async_copy(idx_hbm.at[wid, pl.ds(2 * j, 2)],
                              idx_v.at[pl.ds(2 * ib, 2)], isem[ib]).wait()

    def gather(ib, b):
        pltpu.async_copy(ht_hbm.at[idx_v.at[2 * ib]], rows_v.at[b], gsem[b])

    def gather_wait(ib, b):
        pltpu.make_async_copy(ht_hbm.at[idx_v.at[2 * ib]], rows_v.at[b],
                              gsem[b]).wait()

    def scatter(ib, b):
        pltpu.async_copy(rows_v.at[b], m_sh.at[idx_v.at[2 * ib + 1]],
                         ssem[b], add=True)

    def scatter_wait(ib, b):
        pltpu.make_async_copy(rows_v.at[b], m_sh.at[idx_v.at[2 * ib + 1]],
                              ssem[b]).wait()

    for j in range(IB - 2):
        idx_load(j, j)
    plsc.subcore_barrier()
    for j in range(GLEAD):
        idx_wait(j, j)
        gather(j, j)

    def outer(j0, carry):
        for t in range(IB):
            j = j0 * IB + t
            b = t % NBUF
            ib = t
            gather_wait(ib, b)
            scatter(ib, b)
            b2 = (b + GLEAD) % NBUF
            ib2 = (t + GLEAD) % IB
            ibm2 = (t - SWAIT) % IB
            @pl.when(j >= SWAIT)
            def _():
                scatter_wait(ibm2, b2)
            @pl.when(j + GLEAD < NCH)
            def _():
                idx_wait(j + GLEAD, ib2)
                gather(ib2, b2)
            @pl.when(j + IB - 2 < NCH)
            def _():
                idx_load(j + IB - 2, (t + IB - 2) % IB)
        return carry

    lax.fori_loop(0, NCH // IB, outer, 0)
    for d in range(SWAIT):
        jd = NCH - SWAIT + d
        scatter_wait(jd % IB, jd % NBUF)
    plsc.subcore_barrier()

    pltpu.sync_copy(m_sh.at[pl.ds(s * RPT, RPT)],
                    out_hbm.at[c, pl.ds(s * RPT, RPT)])


def _sc_aggregate(ht4, idx_p, zeros):
    mesh = plsc.VectorSubcoreMesh(core_axis_name="c", subcore_axis_name="s")
    k = pl.kernel(
        _sc_body,
        out_type=jax.ShapeDtypeStruct((NC, NPAD, HID), FP32),
        mesh=mesh,
        scratch_types=[
            pltpu.VMEM((2 * IB, K), I32),
            pltpu.VMEM((NBUF, K, HID), FP32),
            pltpu.VMEM_SHARED((NPAD, HID), FP32),
        ] + [pltpu.SemaphoreType.DMA] * (2 * NBUF + IB),
        name="sc_edge_aggregate",
    )
    return k(ht4, idx_p, zeros)


def _gru_cell(mp_ref, h_ref, wih_ref, whh_ref, bih_ref, bhh_ref):
    m = mp_ref[0] + mp_ref[1]
    h = h_ref[...]
    gi = jnp.dot(m, wih_ref[...], preferred_element_type=FP32) + bih_ref[...]
    gh = jnp.dot(h, whh_ref[...], preferred_element_type=FP32) + bhh_ref[...]
    i_r, i_z, i_n = gi[:, :HID], gi[:, HID:2 * HID], gi[:, 2 * HID:]
    h_r, h_z, h_n = gh[:, :HID], gh[:, HID:2 * HID], gh[:, 2 * HID:]
    r = jax.nn.sigmoid(i_r + h_r)
    z = jax.nn.sigmoid(i_z + h_z)
    n = jnp.tanh(i_n + r * h_n)
    return (1.0 - z) * n + z * h


def _gru_body(mp_ref, h_ref, wih_ref, whh_ref, bih_ref, bhh_ref,
              wcat_ref, h_out, ht_out):
    h_new = _gru_cell(mp_ref, h_ref, wih_ref, whh_ref, bih_ref, bhh_ref)
    h_out[...] = h_new
    ht_out[...] = jnp.dot(h_new, wcat_ref[...], preferred_element_type=FP32)


def _gru(mp, h, wihT, whhT, bih2, bhh2, wcat_next):
    bn = 1000
    grid = N // bn
    in_specs = [
        pl.BlockSpec((NC, bn, HID), lambda i: (0, i, 0)),
        pl.BlockSpec((bn, HID), lambda i: (i, 0)),
        pl.BlockSpec(wihT.shape, lambda i: (0, 0)),
        pl.BlockSpec(whhT.shape, lambda i: (0, 0)),
        pl.BlockSpec(bih2.shape, lambda i: (0, 0)),
        pl.BlockSpec(bhh2.shape, lambda i: (0, 0)),
        pl.BlockSpec(wcat_next.shape, lambda i: (0, 0)),
    ]
    return pl.pallas_call(
        _gru_body,
        grid=(grid,),
        in_specs=in_specs,
        out_specs=[
            pl.BlockSpec((bn, HID), lambda i: (i, 0)),
            pl.BlockSpec((bn, NBT * HID), lambda i: (i, 0)),
        ],
        out_shape=[
            jax.ShapeDtypeStruct((N, HID), FP32),
            jax.ShapeDtypeStruct((N, NBT * HID), FP32),
        ],
    )(mp, h, wihT, whhT, bih2, bhh2, wcat_next)


def _gru_readout_body(mp_ref, h_ref, wih_ref, whh_ref, bih_ref, bhh_ref,
                      batch_ref, fpw_ref, fpb_ref,
                      fc1w_ref, fc1b_ref, bn1g_ref, bn1b_ref,
                      fc2w_ref, fc2b_ref, bn2g_ref, bn2b_ref,
                      ow_ref, ob_ref, o_ref, hs_ref, cnt_ref):
    i = pl.program_id(0)
    nprog = pl.num_programs(0)
    h_new = _gru_cell(mp_ref, h_ref, wih_ref, whh_ref, bih_ref, bhh_ref)
    bn = h_new.shape[0]
    gids = lax.broadcasted_iota(I32, (NG, 1), 0)
    oh = (batch_ref[0] == gids).astype(FP32)

    @pl.when(i == 0)
    def _():
        hs_ref[...] = jnp.zeros_like(hs_ref)
        cnt_ref[...] = jnp.zeros_like(cnt_ref)

    hs_ref[...] += jnp.dot(oh, h_new, preferred_element_type=FP32)
    cnt_ref[...] += jnp.dot(oh, jnp.ones((bn, 128), FP32),
                            preferred_element_type=FP32)

    @pl.when(i == nprog - 1)
    def _():
        hs = hs_ref[...]
        cnt = cnt_ref[:, 0:1]
        g = jnp.dot(hs, fpw_ref[...], preferred_element_type=FP32) \
            + cnt * fpb_ref[...]
        z1 = jax.nn.relu(_bn_eval(
            jnp.dot(g, fc1w_ref[...], preferred_element_type=FP32)
            + fc1b_ref[...], bn1g_ref[...], bn1b_ref[...]))
        z2 = jax.nn.relu(_bn_eval(
            jnp.dot(z1, fc2w_ref[...], preferred_element_type=FP32)
            + fc2b_ref[...], bn2g_ref[...], bn2b_ref[...]))
        o_ref[...] = jnp.dot(z2, ow_ref[...], preferred_element_type=FP32) \
            + ob_ref[...]


def _bn_eval(v, g, b):
    return g * (v / jnp.sqrt(1.0 + BN_EPS)) + b


def _gru_readout(mp, h, wihT, whhT, bih2, bhh2, batch3, p):
    bn = 1000
    grid = N // bn
    consts = (
        p['fp_w'], p['fp_b'].reshape(1, -1),
        p['fc1_w'], p['fc1_b'].reshape(1, -1),
        p['bn1_g'].reshape(1, -1), p['bn1_b'].reshape(1, -1),
        p['fc2_w'], p['fc2_b'].reshape(1, -1),
        p['bn2_g'].reshape(1, -1), p['bn2_b'].reshape(1, -1),
        p['out_w'], p['out_b'].reshape(1, -1),
    )
    in_specs = [
        pl.BlockSpec((NC, bn, HID), lambda i: (0, i, 0)),
        pl.BlockSpec((bn, HID), lambda i: (i, 0)),
        pl.BlockSpec(wihT.shape, lambda i: (0, 0)),
        pl.BlockSpec(whhT.shape, lambda i: (0, 0)),
        pl.BlockSpec(bih2.shape, lambda i: (0, 0)),
        pl.BlockSpec(bhh2.shape, lambda i: (0, 0)),
        pl.BlockSpec((1, 1, bn), lambda i: (i, 0, 0)),
    ]
    in_specs += [pl.BlockSpec(c.shape, lambda i: (0, 0)) for c in consts]
    return pl.pallas_call(
        _gru_readout_body,
        grid=(grid,),
        in_specs=in_specs,
        out_specs=pl.BlockSpec((NG, consts[-2].shape[1]), lambda i: (0, 0)),
        out_shape=jax.ShapeDtypeStruct((NG, consts[-2].shape[1]), FP32),
        scratch_shapes=[
            pltpu.VMEM((NG, HID), FP32),
            pltpu.VMEM((NG, 128), FP32),
        ],
    )(mp, h, wihT, whhT, bih2, bhh2, batch3, *consts)


def _wcat(W):
    return jnp.transpose(W, (2, 0, 1)).reshape(HID, NBT * HID)


def kernel(x, edge_index, edge_attr, batch, params):
    x = x.astype(I32)
    src = edge_index[0].astype(I32)
    dst = edge_index[1].astype(I32)
    bt = edge_attr[:, 0].astype(I32)
    batch3 = batch.astype(I32).reshape(10, 1, 1000)

    p = params
    embs = (p['emb_atomic'], p['emb_degree'], p['emb_aroma'],
            p['emb_fc'], p['emb_hyb'])
    wcats = [_wcat(lp['W']) for lp in p['layers']]

    h, ht, gidx2d = _embed(x, src.reshape(2000, 160), bt.reshape(2000, 160),
                           embs, p['node_proj_w'],
                           p['node_proj_b'].reshape(1, HID), wcats[0])

    gidx_f = gidx2d.reshape(E)
    gidx_p = jnp.concatenate([gidx_f, jnp.zeros((EPAD - E,), I32)])
    dst_p = jnp.concatenate([dst, jnp.full((EPAD - E,), GARB, I32)])
    idx_p = jnp.stack([gidx_p.reshape(NW, NCH, K),
                       dst_p.reshape(NW, NCH, K)],
                      axis=2).reshape(NW, 2 * NCH, K)

    zeros = jnp.zeros((RPT, HID), FP32)
    out = None
    for l, lp in enumerate(p['layers']):
        ht4 = ht.reshape(NBT * N, HID)
        mp = _sc_aggregate(ht4, idx_p, zeros)
        wihT = lp['W_ih'].T
        whhT = lp['W_hh'].T
        bih2 = lp['b_ih'].reshape(1, -1)
        bhh2 = lp['b_hh'].reshape(1, -1)
        if l + 1 < NL:
            h, ht = _gru(mp, h, wihT, whhT, bih2, bhh2, wcats[l + 1])
        else:
            out = _gru_readout(mp, h, wihT, whhT, bih2, bhh2, batch3, p)
    return out
